# Initial kernel scaffold; baseline (speedup 1.0000x reference)
import jax
import jax.numpy as jnp
from jax.experimental import pallas as pl


def kernel(x, edge_index, edge_attr, edge_indices, edge_attrs, edge_indices_f2c, position, node_attrs, clusters, params):
    raise NotImplementedError("write your pallas kernel here")


# speedup vs baseline: 2.5896x; 2.5896x over previous
import functools

import jax
import jax.numpy as jnp
from jax import lax
from jax.experimental import pallas as pl
from jax.experimental.pallas import tpu as pltpu

F32 = jnp.float32
BF16 = jnp.bfloat16
HID = 64


def _elu(x):
    return jnp.where(x > 0, x, jnp.exp(jnp.minimum(x, 0.0)) - 1.0)


def _dot(a, b):
    return jnp.dot(a.astype(BF16), b.astype(BF16), preferred_element_type=F32)


def _dinv_of(deg):
    safe = jnp.where(deg > 0, deg, 1.0)
    return jnp.where(deg > 0, lax.rsqrt(safe), 0.0)


def _edge_mlp_body(e_ref, w1_ref, b1_ref, w2_ref, b2_ref, w3_ref, b3_ref,
                   o_ref):
    e = e_ref[...]
    h = _elu(e * w1_ref[...] + b1_ref[...])
    h = _elu(_dot(h, w2_ref[...]) + b2_ref[...])
    o_ref[...] = _elu(_dot(h, w3_ref[...]) + b3_ref[...])


def _edge_mlp(ea, layers, blk=4096):
    (w1, b1), (w2, b2), (w3, b3) = layers
    e = ea.shape[0]
    grid = (pl.cdiv(e, blk),)
    full = lambda a: pl.BlockSpec(a.shape, lambda i: (0,) * a.ndim)
    return pl.pallas_call(
        _edge_mlp_body,
        grid=grid,
        in_specs=[
            pl.BlockSpec((blk, 1), lambda i: (i, 0)),
            full(w1), pl.BlockSpec((1, HID), lambda i: (0, 0)),
            full(w2), pl.BlockSpec((1, HID), lambda i: (0, 0)),
            full(w3), pl.BlockSpec((1, 1), lambda i: (0, 0)),
        ],
        out_specs=pl.BlockSpec((blk, 1), lambda i: (i, 0)),
        out_shape=jax.ShapeDtypeStruct((e, 1), F32),
    )(ea, w1, b1.reshape(1, HID), w2, b2.reshape(1, HID), w3,
      b3.reshape(1, 1))


def _gcn_dense_body(x_ref, w_ref, deg_ref, o_ref):
    h = _dot(x_ref[...], w_ref[...])
    o_ref[...] = h * _dinv_of(deg_ref[...])


def _gcn_dense(x, w, deg, blk=1024):
    n = x.shape[0]
    return pl.pallas_call(
        _gcn_dense_body,
        grid=(pl.cdiv(n, blk),),
        in_specs=[
            pl.BlockSpec((blk, HID), lambda i: (i, 0)),
            pl.BlockSpec((HID, HID), lambda i: (0, 0)),
            pl.BlockSpec((blk, 1), lambda i: (i, 0)),
        ],
        out_specs=pl.BlockSpec((blk, HID), lambda i: (i, 0)),
        out_shape=jax.ShapeDtypeStruct((n, HID), F32),
    )(x, w, deg)


def _gcn_epi_body(agg_ref, hp_ref, deg_ref, b_ref, o_ref):
    dinv = _dinv_of(deg_ref[...])
    o_ref[...] = _elu(dinv * (agg_ref[...] + hp_ref[...]) + b_ref[...])


def _gcn_epilogue(agg, hp, deg, b, blk=1024):
    n = agg.shape[0]
    return pl.pallas_call(
        _gcn_epi_body,
        grid=(pl.cdiv(n, blk),),
        in_specs=[
            pl.BlockSpec((blk, HID), lambda i: (i, 0)),
            pl.BlockSpec((blk, HID), lambda i: (i, 0)),
            pl.BlockSpec((blk, 1), lambda i: (i, 0)),
            pl.BlockSpec((1, HID), lambda i: (0, 0)),
        ],
        out_specs=pl.BlockSpec((blk, HID), lambda i: (i, 0)),
        out_shape=jax.ShapeDtypeStruct((n, HID), F32),
    )(agg, hp, deg, b.reshape(1, HID))


def _c2f_mlp_body(pc_ref, pf_ref, w1_ref, b1_ref, w2_ref, b2_ref, w3_ref,
                  b3_ref, o_ref):
    d = pc_ref[...] - pf_ref[...]
    w1 = w1_ref[...]
    h = _elu(d[:, 0:1] * w1[0:1, :] + d[:, 1:2] * w1[1:2, :] + b1_ref[...])
    h = _elu(_dot(h, w2_ref[...]) + b2_ref[...])
    o_ref[...] = _elu(_dot(h, w3_ref[...]) + b3_ref[...])


def _c2f_mlp(pc, pf, layers, blk=2048):
    (w1, b1), (w2, b2), (w3, b3) = layers
    e = pc.shape[0]
    return pl.pallas_call(
        _c2f_mlp_body,
        grid=(pl.cdiv(e, blk),),
        in_specs=[
            pl.BlockSpec((blk, 2), lambda i: (i, 0)),
            pl.BlockSpec((blk, 2), lambda i: (i, 0)),
            pl.BlockSpec((2, HID), lambda i: (0, 0)),
            pl.BlockSpec((1, HID), lambda i: (0, 0)),
            pl.BlockSpec((HID, HID), lambda i: (0, 0)),
            pl.BlockSpec((1, HID), lambda i: (0, 0)),
            pl.BlockSpec((HID, HID), lambda i: (0, 0)),
            pl.BlockSpec((1, HID), lambda i: (0, 0)),
        ],
        out_specs=pl.BlockSpec((blk, HID), lambda i: (i, 0)),
        out_shape=jax.ShapeDtypeStruct((e, HID), F32),
    )(pc, pf, w1, b1.reshape(1, HID), w2, b2.reshape(1, HID), w3,
      b3.reshape(1, HID))


def _up_ln_body(ea_ref, xg_ref, w1a_ref, w1b_ref, b1_ref, w2_ref, b2_ref,
                w3_ref, b3_ref, g_ref, bln_ref, o_ref):
    ea = ea_ref[...]
    h = _elu(_dot(ea, w1a_ref[...]) + _dot(xg_ref[...], w1b_ref[...])
             + b1_ref[...])
    h = _elu(_dot(h, w2_ref[...]) + b2_ref[...])
    t = ea + _elu(_dot(h, w3_ref[...]) + b3_ref[...])
    m = jnp.mean(t, axis=-1, keepdims=True)
    c = t - m
    v = jnp.mean(c * c, axis=-1, keepdims=True)
    o_ref[...] = c * lax.rsqrt(v + 1e-5) * g_ref[...] + bln_ref[...]


def _up_ln(ea, xg, layers, g, bln, blk=2048):
    (w1, b1), (w2, b2), (w3, b3) = layers
    e = ea.shape[0]
    hb = pl.BlockSpec((blk, HID), lambda i: (i, 0))
    wfull = pl.BlockSpec((HID, HID), lambda i: (0, 0))
    brow = pl.BlockSpec((1, HID), lambda i: (0, 0))
    return pl.pallas_call(
        _up_ln_body,
        grid=(pl.cdiv(e, blk),),
        in_specs=[hb, hb, wfull, wfull, brow, wfull, brow, wfull, brow,
                  brow, brow],
        out_specs=hb,
        out_shape=jax.ShapeDtypeStruct((e, HID), F32),
    )(ea, xg, w1[:HID], w1[HID:], b1.reshape(1, HID), w2,
      b2.reshape(1, HID), w3, b3.reshape(1, HID), g.reshape(1, HID),
      bln.reshape(1, HID))


def _mean_div_body(s0_ref, c0_ref, o_ref):
    o_ref[...] = s0_ref[...] / jnp.maximum(c0_ref[...], 1.0)


def _mean_div(s, c, blk=1024):
    n = s.shape[0]
    return pl.pallas_call(
        _mean_div_body,
        grid=(pl.cdiv(n, blk),),
        in_specs=[
            pl.BlockSpec((blk, HID), lambda i: (i, 0)),
            pl.BlockSpec((blk, 1), lambda i: (i, 0)),
        ],
        out_specs=pl.BlockSpec((blk, HID), lambda i: (i, 0)),
        out_shape=jax.ShapeDtypeStruct((n, HID), F32),
    )(s, c)


def _node_dec_body(x_ref, w1_ref, b1_ref, w2_ref, b2_ref, w3_ref, b3_ref,
                   o_ref):
    h = _elu(_dot(x_ref[...], w1_ref[...]) + b1_ref[...])
    h = _elu(_dot(h, w2_ref[...]) + b2_ref[...])
    o_ref[...] = _elu(_dot(h, w3_ref[...]) + b3_ref[...])


def _node_dec(x, layers, blk=1024):
    (w1, b1), (w2, b2), (w3, b3) = layers
    n = x.shape[0]
    return pl.pallas_call(
        _node_dec_body,
        grid=(pl.cdiv(n, blk),),
        in_specs=[
            pl.BlockSpec((blk, HID), lambda i: (i, 0)),
            pl.BlockSpec((HID, HID), lambda i: (0, 0)),
            pl.BlockSpec((1, HID), lambda i: (0, 0)),
            pl.BlockSpec((HID, HID), lambda i: (0, 0)),
            pl.BlockSpec((1, HID), lambda i: (0, 0)),
            pl.BlockSpec((HID, 1), lambda i: (0, 0)),
            pl.BlockSpec((1, 1), lambda i: (0, 0)),
        ],
        out_specs=pl.BlockSpec((blk, 1), lambda i: (i, 0)),
        out_shape=jax.ShapeDtypeStruct((n, 1), F32),
    )(x, w1, b1.reshape(1, HID), w2, b2.reshape(1, HID), w3,
      b3.reshape(1, 1))


def _conv_body(x_ref, w_ref, b_ref, o_ref):
    x = x_ref[...]
    z = jnp.zeros((1, 1), F32)
    xl = jnp.concatenate([z, x[:, :-1]], axis=1)
    xr = jnp.concatenate([x[:, 1:], z], axis=1)
    o_ref[...] = (w_ref[0, 0] * xl + w_ref[0, 1] * x + w_ref[0, 2] * xr
                  + b_ref[0, 0])


def _conv3(y, w, b):
    n = y.shape[0]
    x = y.reshape(1, n)
    out = pl.pallas_call(
        _conv_body,
        in_specs=[
            pl.BlockSpec((1, n), lambda: (0, 0)),
            pl.BlockSpec((1, 3), lambda: (0, 0)),
            pl.BlockSpec((1, 1), lambda: (0, 0)),
        ],
        out_specs=pl.BlockSpec((1, n), lambda: (0, 0)),
        out_shape=jax.ShapeDtypeStruct((1, n), F32),
    )(x, w.reshape(1, 3), b.reshape(1, 1))
    return out.reshape(n, 1)


def _seg_sum(vals, idx, n):
    return jax.ops.segment_sum(vals, idx, num_segments=n)


def _mlp_plain(h, layers):
    for w, b in layers:
        h = jax.nn.elu(h @ w + b)
    return h


def _gcn_block(x, row, col, ea, ea_deg, layers, n):
    deg = _seg_sum(ea_deg, col, n) + 1.0
    for w, b in layers:
        hp = _gcn_dense(x, w, deg)
        agg = _seg_sum(ea * hp[row], col, n)
        x = _gcn_epilogue(agg, hp, deg, b)
    return x


def kernel(x, edge_index, edge_attr, edge_indices, edge_attrs,
           edge_indices_f2c, position, node_attrs, clusters, params):
    nc = x.shape[0]
    nf = position.shape[1]

    ea_c = _edge_mlp(edge_attr, params['edge_dec'][0])
    ea_c_deg = _mlp_plain(edge_attr, params['edge_dec'][0])
    x = _gcn_block(x, edge_index[0], edge_index[1], ea_c, ea_c_deg,
                   params['gcn'][0], nc)

    pos_fine = position[0]
    pos_coarse = position[1]
    src = edge_indices_f2c[0, 1]
    dst = edge_indices_f2c[0, 0]
    pc = pos_coarse[src]
    pf = pos_fine[dst]
    ea_c2f = _c2f_mlp(pc, pf, params['c2f'])
    xg = x[clusters[0]]
    t = _up_ln(ea_c2f, xg, params['up'], params['ln_g'], params['ln_b'])
    s = _seg_sum(t, dst, nc)
    c = _seg_sum(jnp.ones((dst.shape[0], 1), F32), dst, nc)
    x_top = _mean_div(s, c)
    x = jnp.concatenate([x_top, jnp.zeros((nf - nc, HID), F32)], axis=0)

    ei = edge_indices[0]
    ea_f = _edge_mlp(edge_attrs[0], params['edge_dec'][1])
    ea_f_deg = _mlp_plain(edge_attrs[0], params['edge_dec'][1])
    x = _gcn_block(x, ei[0], ei[1], ea_f, ea_f_deg, params['gcn'][1], nf)

    y = _node_dec(x, params['node_dec'])
    out = _conv3(y, params['conv_w'], params['conv_b'])
    return (out, ei, ea_f)


# speedup vs baseline: 3.1897x; 1.2318x over previous
import functools

import jax
import jax.numpy as jnp
from jax import lax
from jax.experimental import pallas as pl
from jax.experimental.pallas import tpu as pltpu
from jax.experimental.pallas import tpu_sc as plsc

F32 = jnp.float32
BF16 = jnp.bfloat16
HID = 64


def _elu(x):
    return jnp.where(x > 0, x, jnp.exp(jnp.minimum(x, 0.0)) - 1.0)


def _dot(a, b):
    return jnp.dot(a.astype(BF16), b.astype(BF16), preferred_element_type=F32)


def _dinv_of(deg):
    safe = jnp.where(deg > 0, deg, 1.0)
    return jnp.where(deg > 0, lax.rsqrt(safe), 0.0)


def _edge_mlp_body(e_ref, w1_ref, b1_ref, w2_ref, b2_ref, w3_ref, b3_ref,
                   o_ref):
    e = e_ref[...]
    h = _elu(e * w1_ref[...] + b1_ref[...])
    h = _elu(_dot(h, w2_ref[...]) + b2_ref[...])
    o_ref[...] = _elu(_dot(h, w3_ref[...]) + b3_ref[...])


def _edge_mlp(ea, layers, blk=4096):
    (w1, b1), (w2, b2), (w3, b3) = layers
    e = ea.shape[0]
    grid = (pl.cdiv(e, blk),)
    full = lambda a: pl.BlockSpec(a.shape, lambda i: (0,) * a.ndim)
    return pl.pallas_call(
        _edge_mlp_body,
        grid=grid,
        in_specs=[
            pl.BlockSpec((blk, 1), lambda i: (i, 0)),
            full(w1), pl.BlockSpec((1, HID), lambda i: (0, 0)),
            full(w2), pl.BlockSpec((1, HID), lambda i: (0, 0)),
            full(w3), pl.BlockSpec((1, 1), lambda i: (0, 0)),
        ],
        out_specs=pl.BlockSpec((blk, 1), lambda i: (i, 0)),
        out_shape=jax.ShapeDtypeStruct((e, 1), F32),
    )(ea, w1, b1.reshape(1, HID), w2, b2.reshape(1, HID), w3,
      b3.reshape(1, 1))


def _gcn_dense_body(x_ref, w_ref, deg_ref, o_ref):
    h = _dot(x_ref[...], w_ref[...])
    hp = h * _dinv_of(deg_ref[...])
    o_ref[...] = jnp.concatenate([hp, jnp.zeros_like(hp)], axis=1)


def _gcn_dense(x, w, deg, blk=1024):
    n = x.shape[0]
    return pl.pallas_call(
        _gcn_dense_body,
        grid=(pl.cdiv(n, blk),),
        in_specs=[
            pl.BlockSpec((blk, HID), lambda i: (i, 0)),
            pl.BlockSpec((HID, HID), lambda i: (0, 0)),
            pl.BlockSpec((blk, 1), lambda i: (i, 0)),
        ],
        out_specs=pl.BlockSpec((blk, 2 * HID), lambda i: (i, 0)),
        out_shape=jax.ShapeDtypeStruct((n, 2 * HID), F32),
    )(x, w, deg)


def _gcn_epi_body(agg_ref, hp_ref, deg_ref, b_ref, o_ref):
    dinv = _dinv_of(deg_ref[...])
    hp = hp_ref[...][:, :HID]
    o_ref[...] = _elu(dinv * (agg_ref[...] + hp) + b_ref[...])


def _gcn_epilogue(agg, hp, deg, b, blk=1024):
    n = agg.shape[0]
    return pl.pallas_call(
        _gcn_epi_body,
        grid=(pl.cdiv(n, blk),),
        in_specs=[
            pl.BlockSpec((blk, HID), lambda i: (i, 0)),
            pl.BlockSpec((blk, 2 * HID), lambda i: (i, 0)),
            pl.BlockSpec((blk, 1), lambda i: (i, 0)),
            pl.BlockSpec((1, HID), lambda i: (0, 0)),
        ],
        out_specs=pl.BlockSpec((blk, HID), lambda i: (i, 0)),
        out_shape=jax.ShapeDtypeStruct((n, HID), F32),
    )(agg, hp, deg, b.reshape(1, HID))


def _c2f_mlp_body(pc_ref, pf_ref, w1_ref, b1_ref, w2_ref, b2_ref, w3_ref,
                  b3_ref, o_ref):
    d = pc_ref[...] - pf_ref[...]
    w1 = w1_ref[...]
    h = _elu(d[:, 0:1] * w1[0:1, :] + d[:, 1:2] * w1[1:2, :] + b1_ref[...])
    h = _elu(_dot(h, w2_ref[...]) + b2_ref[...])
    o_ref[...] = _elu(_dot(h, w3_ref[...]) + b3_ref[...])


def _c2f_mlp(pc, pf, layers, blk=2048):
    (w1, b1), (w2, b2), (w3, b3) = layers
    e = pc.shape[0]
    return pl.pallas_call(
        _c2f_mlp_body,
        grid=(pl.cdiv(e, blk),),
        in_specs=[
            pl.BlockSpec((blk, 2), lambda i: (i, 0)),
            pl.BlockSpec((blk, 2), lambda i: (i, 0)),
            pl.BlockSpec((2, HID), lambda i: (0, 0)),
            pl.BlockSpec((1, HID), lambda i: (0, 0)),
            pl.BlockSpec((HID, HID), lambda i: (0, 0)),
            pl.BlockSpec((1, HID), lambda i: (0, 0)),
            pl.BlockSpec((HID, HID), lambda i: (0, 0)),
            pl.BlockSpec((1, HID), lambda i: (0, 0)),
        ],
        out_specs=pl.BlockSpec((blk, HID), lambda i: (i, 0)),
        out_shape=jax.ShapeDtypeStruct((e, HID), F32),
    )(pc, pf, w1, b1.reshape(1, HID), w2, b2.reshape(1, HID), w3,
      b3.reshape(1, HID))


def _up_ln_body(ea_ref, xg_ref, w1a_ref, w1b_ref, b1_ref, w2_ref, b2_ref,
                w3_ref, b3_ref, g_ref, bln_ref, o_ref):
    ea = ea_ref[...]
    h = _elu(_dot(ea, w1a_ref[...]) + _dot(xg_ref[...], w1b_ref[...])
             + b1_ref[...])
    h = _elu(_dot(h, w2_ref[...]) + b2_ref[...])
    t = ea + _elu(_dot(h, w3_ref[...]) + b3_ref[...])
    m = jnp.mean(t, axis=-1, keepdims=True)
    c = t - m
    v = jnp.mean(c * c, axis=-1, keepdims=True)
    o_ref[...] = c * lax.rsqrt(v + 1e-5) * g_ref[...] + bln_ref[...]


def _up_ln(ea, xg, layers, g, bln, blk=2048):
    (w1, b1), (w2, b2), (w3, b3) = layers
    e = ea.shape[0]
    hb = pl.BlockSpec((blk, HID), lambda i: (i, 0))
    wfull = pl.BlockSpec((HID, HID), lambda i: (0, 0))
    brow = pl.BlockSpec((1, HID), lambda i: (0, 0))
    return pl.pallas_call(
        _up_ln_body,
        grid=(pl.cdiv(e, blk),),
        in_specs=[hb, hb, wfull, wfull, brow, wfull, brow, wfull, brow,
                  brow, brow],
        out_specs=hb,
        out_shape=jax.ShapeDtypeStruct((e, HID), F32),
    )(ea, xg, w1[:HID], w1[HID:], b1.reshape(1, HID), w2,
      b2.reshape(1, HID), w3, b3.reshape(1, HID), g.reshape(1, HID),
      bln.reshape(1, HID))


def _mean_div_body(s0_ref, c0_ref, o_ref):
    o_ref[...] = s0_ref[...] / jnp.maximum(c0_ref[...], 1.0)


def _mean_div(s, c, blk=1024):
    n = s.shape[0]
    return pl.pallas_call(
        _mean_div_body,
        grid=(pl.cdiv(n, blk),),
        in_specs=[
            pl.BlockSpec((blk, HID), lambda i: (i, 0)),
            pl.BlockSpec((blk, 1), lambda i: (i, 0)),
        ],
        out_specs=pl.BlockSpec((blk, HID), lambda i: (i, 0)),
        out_shape=jax.ShapeDtypeStruct((n, HID), F32),
    )(s, c)


def _node_dec_body(x_ref, w1_ref, b1_ref, w2_ref, b2_ref, w3_ref, b3_ref,
                   o_ref):
    h = _elu(_dot(x_ref[...], w1_ref[...]) + b1_ref[...])
    h = _elu(_dot(h, w2_ref[...]) + b2_ref[...])
    o_ref[...] = _elu(_dot(h, w3_ref[...]) + b3_ref[...])


def _node_dec(x, layers, blk=1024):
    (w1, b1), (w2, b2), (w3, b3) = layers
    n = x.shape[0]
    return pl.pallas_call(
        _node_dec_body,
        grid=(pl.cdiv(n, blk),),
        in_specs=[
            pl.BlockSpec((blk, HID), lambda i: (i, 0)),
            pl.BlockSpec((HID, HID), lambda i: (0, 0)),
            pl.BlockSpec((1, HID), lambda i: (0, 0)),
            pl.BlockSpec((HID, HID), lambda i: (0, 0)),
            pl.BlockSpec((1, HID), lambda i: (0, 0)),
            pl.BlockSpec((HID, 1), lambda i: (0, 0)),
            pl.BlockSpec((1, 1), lambda i: (0, 0)),
        ],
        out_specs=pl.BlockSpec((blk, 1), lambda i: (i, 0)),
        out_shape=jax.ShapeDtypeStruct((n, 1), F32),
    )(x, w1, b1.reshape(1, HID), w2, b2.reshape(1, HID), w3,
      b3.reshape(1, 1))


def _conv_body(x_ref, w_ref, b_ref, o_ref):
    x = x_ref[...]
    z = jnp.zeros((1, 1), F32)
    xl = jnp.concatenate([z, x[:, :-1]], axis=1)
    xr = jnp.concatenate([x[:, 1:], z], axis=1)
    o_ref[...] = (w_ref[0, 0] * xl + w_ref[0, 1] * x + w_ref[0, 2] * xr
                  + b_ref[0, 0])


def _conv3(y, w, b):
    n = y.shape[0]
    x = y.reshape(1, n)
    out = pl.pallas_call(
        _conv_body,
        in_specs=[
            pl.BlockSpec((1, n), lambda: (0, 0)),
            pl.BlockSpec((1, 3), lambda: (0, 0)),
            pl.BlockSpec((1, 1), lambda: (0, 0)),
        ],
        out_specs=pl.BlockSpec((1, n), lambda: (0, 0)),
        out_shape=jax.ShapeDtypeStruct((1, n), F32),
    )(x, w.reshape(1, 3), b.reshape(1, 1))
    return out.reshape(n, 1)


_SC_TILES = 16
_CHUNK = 80


def _sc_gcn_agg(hp, row, col, ew, n_out, npass=1):
    e = row.shape[0]
    nchunks = e // _CHUNK
    cpt = nchunks // _SC_TILES
    assert nchunks * _CHUNK == e and cpt * _SC_TILES == nchunks
    rsize = (-(-n_out // (2 * npass)) + 7) // 8 * 8
    nz = rsize // 8
    nzl = (nz + _SC_TILES - 1) // _SC_TILES
    mesh = plsc.VectorSubcoreMesh(core_axis_name="c", subcore_axis_name="s")

    @functools.partial(
        pl.kernel,
        out_type=jax.ShapeDtypeStruct((n_out, HID), F32),
        mesh=mesh,
        scratch_types=[
            pltpu.VMEM((_CHUNK,), jnp.int32),
            pltpu.VMEM((_CHUNK,), jnp.int32),
            pltpu.VMEM((_CHUNK,), F32),
            pltpu.VMEM((_CHUNK,), F32),
            pltpu.VMEM((_CHUNK,), jnp.int32),
            pltpu.VMEM((_CHUNK, 2 * HID), F32),
            pltpu.VMEM((_CHUNK, HID), F32),
            pltpu.VMEM((8, HID), F32),
            pltpu.VMEM_SHARED((rsize, HID), F32),
            pltpu.SemaphoreType.DMA,
        ],
    )
    def k(hp_hbm, row_hbm, col_hbm, ew_hbm, out_hbm,
          row_v, col_v, ew_v, ow_v, dst_v, gat_v, msg_v, zero_v, acc, sem):
        core = lax.axis_index("c")
        tile = lax.axis_index("s")

        @pl.loop(0, 8)
        def _(r):
            for q in range(4):
                zero_v[r, pl.ds(q * 16, 16)] = jnp.zeros((16,), F32)

        @pl.loop(0, npass)
        def _(p):
            base_node = (core * npass + p) * rsize

            @pl.loop(0, nzl)
            def _(j):
                i = j * _SC_TILES + tile

                @pl.when(i < nz)
                def _():
                    pltpu.sync_copy(zero_v, acc.at[pl.ds(i * 8, 8)])

            plsc.subcore_barrier()

            @pl.loop(0, cpt)
            def _(j):
                off = (tile * cpt + j) * _CHUNK
                pltpu.sync_copy(row_hbm.at[pl.ds(off, _CHUNK)], row_v)
                gcp = pltpu.async_copy(hp_hbm.at[row_v], gat_v, sem)
                pltpu.sync_copy(col_hbm.at[pl.ds(off, _CHUNK)], col_v)
                pltpu.sync_copy(ew_hbm.at[pl.ds(off, _CHUNK)], ew_v)

                @pl.loop(0, _CHUNK // 16)
                def _(g):
                    sl = pl.ds(g * 16, 16)
                    loc = col_v[sl] - base_node
                    ok = (loc >= 0) & (loc < rsize)
                    dst_v[sl] = jnp.where(ok, loc, 0)
                    ow_v[sl] = jnp.where(ok, ew_v[sl], 0.0)

                gcp.wait()

                @pl.loop(0, _CHUNK // 16)
                def _(g):
                    ow16 = ow_v[pl.ds(g * 16, 16)]
                    for jj in range(16):
                        cvec = ow16.at[jnp.full((16,), jj, jnp.int32)].get(
                            mode="promise_in_bounds")
                        r = g * 16 + jj
                        for q in range(4):
                            sl = pl.ds(q * 16, 16)
                            msg_v[r, sl] = gat_v[r, sl] * cvec

                pltpu.sync_copy(msg_v, acc.at[dst_v], add=True)

            plsc.subcore_barrier()

            @pl.loop(0, nzl)
            def _(j):
                i = j * _SC_TILES + tile

                @pl.when((i < nz) & (base_node + i * 8 + 8 <= n_out))
                def _():
                    pltpu.sync_copy(
                        acc.at[pl.ds(i * 8, 8)],
                        out_hbm.at[pl.ds(base_node + i * 8, 8)])

            plsc.subcore_barrier()

    return k(hp, row, col, ew)


def _seg_sum(vals, idx, n):
    return jax.ops.segment_sum(vals, idx, num_segments=n)


def _mlp_plain(h, layers):
    for w, b in layers:
        h = jax.nn.elu(h @ w + b)
    return h


def _gcn_block(x, row, col, ea, ea_deg, layers, n, npass=1):
    deg = _seg_sum(ea_deg, col, n) + 1.0
    ew = ea.reshape(-1)
    ws = jnp.stack([w for w, _ in layers])
    bs = jnp.stack([b for _, b in layers])

    def body(xc, wb):
        w, b = wb
        hp = _gcn_dense(xc, w, deg)
        agg = _sc_gcn_agg(hp, row, col, ew, n, npass)
        return _gcn_epilogue(agg, hp, deg, b), None

    x, _ = lax.scan(body, x, (ws, bs))
    return x


def kernel(x, edge_index, edge_attr, edge_indices, edge_attrs,
           edge_indices_f2c, position, node_attrs, clusters, params):
    nc = x.shape[0]
    nf = position.shape[1]

    ea_c = _edge_mlp(edge_attr, params['edge_dec'][0])
    ea_c_deg = _mlp_plain(edge_attr, params['edge_dec'][0])
    x = _gcn_block(x, edge_index[0], edge_index[1], ea_c, ea_c_deg,
                   params['gcn'][0], nc, npass=2)

    pos_fine = position[0]
    pos_coarse = position[1]
    src = edge_indices_f2c[0, 1]
    dst = edge_indices_f2c[0, 0]
    pc = pos_coarse[src]
    pf = pos_fine[dst]
    ea_c2f = _c2f_mlp(pc, pf, params['c2f'])
    xg = x[clusters[0]]
    t = _up_ln(ea_c2f, xg, params['up'], params['ln_g'], params['ln_b'])
    s = _seg_sum(t, dst, nc)
    c = _seg_sum(jnp.ones((dst.shape[0], 1), F32), dst, nc)
    x_top = _mean_div(s, c)
    x = jnp.concatenate([x_top, jnp.zeros((nf - nc, HID), F32)], axis=0)

    ei = edge_indices[0]
    ea_f = _edge_mlp(edge_attrs[0], params['edge_dec'][1])
    ea_f_deg = _mlp_plain(edge_attrs[0], params['edge_dec'][1])
    x = _gcn_block(x, ei[0], ei[1], ea_f, ea_f_deg, params['gcn'][1], nf,
                   npass=2)

    y = _node_dec(x, params['node_dec'])
    out = _conv3(y, params['conv_w'], params['conv_b'])
    return (out, ei, ea_f)
